# baseline (speedup 1.0000x reference)
import functools

import jax
import jax.numpy as jnp
from jax import lax
from jax.experimental import pallas as pl
from jax.experimental.pallas import tpu as pltpu
from jax.experimental.pallas import tpu_sc as plsc

F32 = jnp.float32
I32 = jnp.int32

N = 1000
NP = 1024
HID = 64
NG = 4
HOPS = 4
ALPHA = 0.15
BATCH = 256
E = 32000
EP = 32768
NS = 16
EDGES_PER_SUB = EP // NS
E_PER_SUB = E // NS
ECHUNKS = EDGES_PER_SUB // 128
ROWS_PER_CORE = 512
ADJ_ROWS = 528
ADJ_WORDS = ADJ_ROWS * NP
ADJ_WORDS_PER_SUB = ADJ_WORDS // NS
ADJ_ZCHUNK = ADJ_WORDS_PER_SUB // 8
OUT_WORDS_PER_CORE = ROWS_PER_CORE * NP
A_ROWS_PER_SUB = 8
A_WORDS_PER_SUB = A_ROWS_PER_SUB * NP
A_WORDS_PER_CORE = A_WORDS_PER_SUB * NS
A_ZCHUNK = A_WORDS_PER_SUB // 4

_MESH = plsc.VectorSubcoreMesh(core_axis_name="c", subcore_axis_name="s")
_SC_PARAMS = pltpu.CompilerParams(needs_layout_passes=False)
_HI = jax.lax.Precision.HIGHEST


def _adj_body(src_hbm, dst_hbm, uv_hbm, out_hbm,
              uv_v, den_v, src_v, dst_v, ex_v, att_v,
              didx, aidx, zbuf, den_sh, adj_sh,
              sem_stage, sem_zero, sem_scat):
    c = lax.axis_index("c")
    s = lax.axis_index("s")

    ebase = s * E_PER_SUB
    d_uv = pltpu.async_copy(uv_hbm, uv_v, sem_stage)
    d_src = pltpu.async_copy(src_hbm.at[pl.ds(ebase, E_PER_SUB)],
                             src_v.at[pl.ds(0, E_PER_SUB)], sem_stage)
    d_dst = pltpu.async_copy(dst_hbm.at[pl.ds(ebase, E_PER_SUB)],
                             dst_v.at[pl.ds(0, E_PER_SUB)], sem_stage)

    z16 = jnp.zeros((16,), F32)

    def _zb(i, _):
        zbuf[pl.ds(i * 16, 16)] = z16
        return None
    lax.fori_loop(0, ADJ_ZCHUNK // 16, _zb, None)

    zds = [pltpu.async_copy(
        zbuf,
        adj_sh.at[pl.ds(s * ADJ_WORDS_PER_SUB + t * ADJ_ZCHUNK, ADJ_ZCHUNK)],
        sem_zero) for t in range(8)]

    @pl.when(s == 0)
    def _():
        pltpu.sync_copy(zbuf.at[pl.ds(0, NP)], den_sh)

    plsc.subcore_barrier()

    d_uv.wait()
    d_src.wait()
    d_dst.wait()

    pad16 = jnp.full((16,), NP - 1, I32)
    for t in range(3):
        src_v[pl.ds(E_PER_SUB + t * 16, 16)] = pad16
        dst_v[pl.ds(E_PER_SUB + t * 16, 16)] = pad16

    cbase = c * ROWS_PER_CORE

    p1 = []
    for J in range(ECHUNKS):
        def _p1(i, _):
            o = J * 128 + i * 16
            s16 = src_v[pl.ds(o, 16)] & (NP - 1)
            d16 = dst_v[pl.ds(o, 16)] & (NP - 1)
            us = plsc.load_gather(uv_v, [s16 + s16])
            vs = plsc.load_gather(uv_v, [d16 + d16 + 1])
            e = us + vs
            e = jnp.where(e < 0.0, e * 0.2, e)
            ex_v[pl.ds(o, 16)] = jnp.exp(e)
            lidx = d16 - cbase
            inr = (lidx >= 0) & (lidx < ROWS_PER_CORE)
            dn = jnp.where(inr, lidx,
                           ROWS_PER_CORE + lax.shift_right_logical(s16, 1))
            didx[J, pl.ds(i * 16, 16)] = dn
            arow = jnp.where(inr, lidx,
                             ROWS_PER_CORE + lax.shift_right_logical(s16, 6))
            aidx[J, pl.ds(i * 16, 16)] = arow * NP + s16
            return None
        lax.fori_loop(0, 8, _p1, None)
        p1.append(pltpu.async_copy(ex_v.at[pl.ds(J * 128, 128)],
                                   den_sh.at[didx.at[J]], sem_scat, add=True))

    for d in zds:
        d.wait()
    for d in p1:
        d.wait()
    plsc.subcore_barrier()

    pltpu.sync_copy(den_sh, den_v)
    p2 = []
    for J in range(ECHUNKS):
        def _p2(i, _):
            o = J * 128 + i * 16
            dn = didx[J, pl.ds(i * 16, 16)]
            dv = plsc.load_gather(den_v, [dn])
            att_v[pl.ds(o, 16)] = ex_v[pl.ds(o, 16)] / (dv + 1e-16)
            return None
        lax.fori_loop(0, 8, _p2, None)
        p2.append(pltpu.async_copy(att_v.at[pl.ds(J * 128, 128)],
                                   adj_sh.at[aidx.at[J]], sem_scat, add=True))
    for d in p2:
        d.wait()
    plsc.subcore_barrier()

    ods = []
    for k in range(32):
        lr = s * 32 + k
        ods.append(pltpu.async_copy(adj_sh.at[pl.ds(lr * NP, NP)],
                                    out_hbm.at[c * ROWS_PER_CORE + lr],
                                    sem_stage))
    for d in ods:
        d.wait()


_adj_call = functools.partial(
    pl.kernel,
    out_type=jax.ShapeDtypeStruct((NP, NP), F32),
    mesh=_MESH,
    compiler_params=_SC_PARAMS,
    scratch_types=[
        pltpu.VMEM((2 * NP,), F32),
        pltpu.VMEM((NP,), F32),
        pltpu.VMEM((EDGES_PER_SUB,), I32),
        pltpu.VMEM((EDGES_PER_SUB,), I32),
        pltpu.VMEM((EDGES_PER_SUB,), F32),
        pltpu.VMEM((EDGES_PER_SUB,), F32),
        pltpu.VMEM((ECHUNKS, 128), I32),
        pltpu.VMEM((ECHUNKS, 128), I32),
        pltpu.VMEM((ADJ_ZCHUNK,), F32),
        pltpu.VMEM_SHARED((NP,), F32),
        pltpu.VMEM_SHARED((ADJ_WORDS,), F32),
        pltpu.SemaphoreType.DMA,
        pltpu.SemaphoreType.DMA,
        pltpu.SemaphoreType.DMA,
    ],
)(_adj_body)


def _adjhist_body(src_hbm, dst_hbm, uv_hbm, cate_hbm, wp_hbm,
                  out_hbm, a_out_hbm,
                  uv_v, den_v, src_v, dst_v, ex_v, att_v, didx, aidx, zbuf,
                  wp_v, cl_v, idxb, den_sh, adj_sh, a_sh,
                  sem_stage, sem_zero, sem_scat,
                  sem_stage2, sem_zero2, sem_scat2):
    c = lax.axis_index("c")
    s = lax.axis_index("s")

    ebase = s * E_PER_SUB
    d_uv = pltpu.async_copy(uv_hbm, uv_v, sem_stage)
    d_src = pltpu.async_copy(src_hbm.at[pl.ds(ebase, E_PER_SUB)],
                             src_v.at[pl.ds(0, E_PER_SUB)], sem_stage)
    d_dst = pltpu.async_copy(dst_hbm.at[pl.ds(ebase, E_PER_SUB)],
                             dst_v.at[pl.ds(0, E_PER_SUB)], sem_stage)
    d_wp = pltpu.async_copy(wp_hbm, wp_v.at[pl.ds(0, N)], sem_stage2)
    rows0 = c * (NS * A_ROWS_PER_SUB) + s * A_ROWS_PER_SUB
    d_cl = pltpu.async_copy(cate_hbm.at[pl.ds(rows0, A_ROWS_PER_SUB)], cl_v,
                            sem_stage2)

    z16 = jnp.zeros((16,), F32)

    def _zb(i, _):
        zbuf[pl.ds(i * 16, 16)] = z16
        return None
    lax.fori_loop(0, ADJ_ZCHUNK // 16, _zb, None)

    zds = [pltpu.async_copy(
        zbuf,
        adj_sh.at[pl.ds(s * ADJ_WORDS_PER_SUB + t * ADJ_ZCHUNK, ADJ_ZCHUNK)],
        sem_zero) for t in range(8)]
    zda = [pltpu.async_copy(
        zbuf.at[pl.ds(0, A_ZCHUNK)],
        a_sh.at[pl.ds(s * A_WORDS_PER_SUB + t * A_ZCHUNK, A_ZCHUNK)],
        sem_zero2) for t in range(4)]

    @pl.when(s == 0)
    def _():
        pltpu.sync_copy(zbuf.at[pl.ds(0, NP)], den_sh)

    plsc.subcore_barrier()

    d_wp.wait()
    d_cl.wait()
    for d in zda:
        d.wait()
    wp_v[pl.ds(1008, 16)] = z16
    lanes = lax.iota(I32, 16) + 992
    w16 = wp_v[pl.ds(992, 16)]
    wp_v[pl.ds(992, 16)] = jnp.where(lanes < N, w16, 0.0)

    for r in range(A_ROWS_PER_SUB):
        rbase = s * A_WORDS_PER_SUB + r * NP
        for j in range(8):
            def _fill(i, _):
                cv = cl_v[r, pl.ds(j * 128 + i * 16, 16)] & (NP - 1)
                idxb[r * 8 + j, pl.ds(i * 16, 16)] = cv + rbase
                return None
            lax.fori_loop(0, 8, _fill, None)

    hd = []
    for r in range(A_ROWS_PER_SUB):
        for j in range(8):
            hd.append(pltpu.async_copy(wp_v.at[pl.ds(j * 128, 128)],
                                       a_sh.at[idxb.at[r * 8 + j]],
                                       sem_scat2, add=True))
        if r >= 1:
            for d in hd[(r - 1) * 8:r * 8]:
                d.wait()
    for d in hd[(A_ROWS_PER_SUB - 1) * 8:]:
        d.wait()
    oda = [pltpu.async_copy(
        a_sh.at[pl.ds((s * A_ROWS_PER_SUB + r) * NP, NP)],
        a_out_hbm.at[rows0 + r], sem_stage2)
        for r in range(A_ROWS_PER_SUB)]

    d_uv.wait()
    d_src.wait()
    d_dst.wait()
    pad16 = jnp.full((16,), NP - 1, I32)
    for t in range(3):
        src_v[pl.ds(E_PER_SUB + t * 16, 16)] = pad16
        dst_v[pl.ds(E_PER_SUB + t * 16, 16)] = pad16

    cbase = c * ROWS_PER_CORE
    p1 = []
    for J in range(ECHUNKS):
        def _p1(i, _):
            o = J * 128 + i * 16
            s16 = src_v[pl.ds(o, 16)] & (NP - 1)
            d16 = dst_v[pl.ds(o, 16)] & (NP - 1)
            us = plsc.load_gather(uv_v, [s16 + s16])
            vs = plsc.load_gather(uv_v, [d16 + d16 + 1])
            e = us + vs
            e = jnp.where(e < 0.0, e * 0.2, e)
            ex_v[pl.ds(o, 16)] = jnp.exp(e)
            lidx = d16 - cbase
            inr = (lidx >= 0) & (lidx < ROWS_PER_CORE)
            dn = jnp.where(inr, lidx,
                           ROWS_PER_CORE + lax.shift_right_logical(s16, 1))
            didx[J, pl.ds(i * 16, 16)] = dn
            arow = jnp.where(inr, lidx,
                             ROWS_PER_CORE + lax.shift_right_logical(s16, 6))
            aidx[J, pl.ds(i * 16, 16)] = arow * NP + s16
            return None
        lax.fori_loop(0, 8, _p1, None)
        p1.append(pltpu.async_copy(ex_v.at[pl.ds(J * 128, 128)],
                                   den_sh.at[didx.at[J]], sem_scat, add=True))
    for d in zds:
        d.wait()
    for d in p1:
        d.wait()
    plsc.subcore_barrier()

    pltpu.sync_copy(den_sh, den_v)
    p2 = []
    for J in range(ECHUNKS):
        def _p2(i, _):
            o = J * 128 + i * 16
            dn = didx[J, pl.ds(i * 16, 16)]
            dv = plsc.load_gather(den_v, [dn])
            att_v[pl.ds(o, 16)] = ex_v[pl.ds(o, 16)] / (dv + 1e-16)
            return None
        lax.fori_loop(0, 8, _p2, None)
        p2.append(pltpu.async_copy(att_v.at[pl.ds(J * 128, 128)],
                                   adj_sh.at[aidx.at[J]], sem_scat, add=True))
    for d in p2:
        d.wait()
    plsc.subcore_barrier()

    ods = []
    for k in range(32):
        lr = s * 32 + k
        ods.append(pltpu.async_copy(adj_sh.at[pl.ds(lr * NP, NP)],
                                    out_hbm.at[c * ROWS_PER_CORE + lr],
                                    sem_stage))
    for d in ods:
        d.wait()
    for d in oda:
        d.wait()


_adjhist_call = functools.partial(
    pl.kernel,
    out_type=(jax.ShapeDtypeStruct((NP, NP), F32),
              jax.ShapeDtypeStruct((BATCH, NP), F32)),
    mesh=_MESH,
    compiler_params=_SC_PARAMS,
    scratch_types=[
        pltpu.VMEM((2 * NP,), F32),
        pltpu.VMEM((NP,), F32),
        pltpu.VMEM((EDGES_PER_SUB,), I32),
        pltpu.VMEM((EDGES_PER_SUB,), I32),
        pltpu.VMEM((EDGES_PER_SUB,), F32),
        pltpu.VMEM((EDGES_PER_SUB,), F32),
        pltpu.VMEM((ECHUNKS, 128), I32),
        pltpu.VMEM((ECHUNKS, 128), I32),
        pltpu.VMEM((ADJ_ZCHUNK,), F32),
        pltpu.VMEM((NP,), F32),
        pltpu.VMEM((A_ROWS_PER_SUB, NP), I32),
        pltpu.VMEM((64, 128), I32),
        pltpu.VMEM_SHARED((NP,), F32),
        pltpu.VMEM_SHARED((ADJ_WORDS,), F32),
        pltpu.VMEM_SHARED((A_WORDS_PER_CORE,), F32),
        pltpu.SemaphoreType.DMA,
        pltpu.SemaphoreType.DMA,
        pltpu.SemaphoreType.DMA,
        pltpu.SemaphoreType.DMA,
        pltpu.SemaphoreType.DMA,
        pltpu.SemaphoreType.DMA,
    ],
)(_adjhist_body)


def _dot(a, b):
    return jnp.dot(a, b, precision=_HI, preferred_element_type=F32)


def _hops_ln(adj, ht, h, g, b):
    bf = jnp.bfloat16
    ah = adj.astype(bf)
    al = (adj - ah.astype(F32)).astype(bf)

    def _mm3(zz):
        zh = zz.astype(bf)
        zl = (zz - zh.astype(F32)).astype(bf)
        acc = jnp.dot(ah, zh, preferred_element_type=F32)
        acc += jnp.dot(ah, zl, preferred_element_type=F32)
        acc += jnp.dot(al, zh, preferred_element_type=F32)
        return acc

    z = ht
    for _ in range(HOPS):
        z = (1.0 - ALPHA) * _mm3(z) + ALPHA * ht
    o = h + z
    mu = jnp.mean(o, axis=-1, keepdims=True)
    var = jnp.mean((o - mu) * (o - mu), axis=-1, keepdims=True)
    ln = g * (o - mu) * lax.rsqrt(var + 1e-5) + b
    rows = lax.broadcasted_iota(I32, (NP, HID), 0)
    return jnp.where(rows < N, ln, 0.0)


def _pad_rows(h):
    return jnp.concatenate([h, jnp.zeros((NP - N, HID), F32)], axis=0)


def _prep_body(h_ref, w_ref, a2_ref, ht_ref, uv_ref):
    ht = _dot(_pad_rows(h_ref[...]), w_ref[...])
    ht_ref[...] = ht
    uv_ref[...] = _dot(ht, a2_ref[...])


def _prep(h, w, a2):
    return pl.pallas_call(
        _prep_body,
        out_shape=[jax.ShapeDtypeStruct((NP, HID), F32),
                   jax.ShapeDtypeStruct((NP, 2), F32)],
    )(h, w, a2)


def _mid_body(adj_ref, ht_ref, h_ref, g_ref, b_ref, w2_ref, a2_ref,
              h2_ref, ht2_ref, uv2_ref):
    h2 = _hops_ln(adj_ref[...], ht_ref[...], _pad_rows(h_ref[...]),
                  g_ref[...], b_ref[...])
    h2_ref[...] = h2
    ht2 = _dot(h2, w2_ref[...])
    ht2_ref[...] = ht2
    uv2_ref[...] = _dot(ht2, a2_ref[...])


def _mid(adj, ht, h, g, b, w2, a2):
    return pl.pallas_call(
        _mid_body,
        out_shape=[jax.ShapeDtypeStruct((NP, HID), F32),
                   jax.ShapeDtypeStruct((NP, HID), F32),
                   jax.ShapeDtypeStruct((NP, 2), F32)],
    )(adj, ht, h, g, b, w2, a2)


def _tail_body(adj_ref, ht_ref, h_ref, g_ref, b_ref, wg_ref, bg_ref,
               gam_ref, bet_ref, wp_ref, bp_ref, a_ref, o_ref):
    gph = _hops_ln(adj_ref[...], ht_ref[...], h_ref[...], g_ref[...],
                   b_ref[...])
    logits = _dot(gph, wg_ref[...]) + bg_ref[...]
    mx = jnp.max(logits, axis=1, keepdims=True)
    exl = jnp.exp(logits - mx)
    score = exl / jnp.sum(exl, axis=1, keepdims=True)

    mu = jnp.sum(gph, axis=0, keepdims=True) / float(NG * N)
    s2 = jnp.sum(score * score, axis=1, keepdims=True)
    msq = jnp.sum(gph * gph * s2, axis=0, keepdims=True) / float(NG * N)
    rstd = lax.rsqrt(msq - mu * mu + 1e-5)

    scale = gam_ref[...] * rstd
    rows = lax.broadcasted_iota(I32, (NP, 1), 0)
    keep = (rows > 0) & (rows < N)
    m = jnp.where(keep, gph * scale, 0.0)
    mp = jnp.concatenate([score[:, g:g + 1] * m for g in range(NG)], axis=1)

    sw = jnp.sum(wp_ref[...])
    cvec = (bet_ref[...] - gam_ref[...] * mu * rstd) * sw + bp_ref[...]
    cv = jnp.concatenate([cvec] * NG, axis=1)
    o_ref[...] = _dot(a_ref[...], mp) + cv


def _tail(adj, ht, h, g, b, wg, bg, gam, bet, wp, bp, a):
    return pl.pallas_call(
        _tail_body,
        out_shape=jax.ShapeDtypeStruct((BATCH, NG * HID), F32),
    )(adj, ht, h, g, b, wg, bg, gam, bet, wp, bp, a)


def kernel(cate_list, edge_index, emb, W1, a_src1, a_dst1, ln_g1, ln_b1,
           W2, a_src2, a_dst2, ln_g2, ln_b2, Wg, bg, Wpool, bpool,
           bn_gamma, bn_beta):
    cate_p = jnp.pad(cate_list, ((0, 0), (0, NP - N)))

    src = edge_index[0]
    dst = edge_index[1]
    ht1, uv1 = _prep(emb, W1, jnp.stack([a_src1, a_dst1], axis=1))
    adj1, a_mat = _adjhist_call(src, dst, uv1.reshape(-1), cate_p, Wpool)
    h2, ht2, uv2 = _mid(adj1, ht1, emb,
                        ln_g1.reshape(1, HID), ln_b1.reshape(1, HID),
                        W2, jnp.stack([a_src2, a_dst2], axis=1))

    adj2 = _adj_call(src, dst, uv2.reshape(-1))
    out = _tail(adj2, ht2, h2,
                ln_g2.reshape(1, HID), ln_b2.reshape(1, HID),
                Wg, bg.reshape(1, NG), bn_gamma.reshape(1, HID),
                bn_beta.reshape(1, HID), Wpool.reshape(1, N),
                bpool.reshape(1, 1), a_mat)
    return out.reshape(BATCH, NG, HID)


# speedup vs baseline: 1.0806x; 1.0806x over previous
import functools

import jax
import jax.numpy as jnp
from jax import lax
from jax.experimental import pallas as pl
from jax.experimental.pallas import tpu as pltpu
from jax.experimental.pallas import tpu_sc as plsc

F32 = jnp.float32
I32 = jnp.int32

N = 1000
NP = 1024
HID = 64
NG = 4
HOPS = 4
ALPHA = 0.15
BATCH = 256
E = 32000
EP = 32768
NS = 16
EDGES_PER_SUB = EP // NS
E_PER_SUB = E // NS
ECHUNKS = EDGES_PER_SUB // 128
ROWS_PER_CORE = 512
ADJ_ROWS = 528
ADJ_WORDS = ADJ_ROWS * NP
ADJ_WORDS_PER_SUB = ADJ_WORDS // NS
ADJ_ZCHUNK = ADJ_WORDS_PER_SUB // 8
OUT_WORDS_PER_CORE = ROWS_PER_CORE * NP
A_ROWS_PER_SUB = 8
A_WORDS_PER_SUB = A_ROWS_PER_SUB * NP
A_WORDS_PER_CORE = A_WORDS_PER_SUB * NS
A_ZCHUNK = A_WORDS_PER_SUB // 4

_MESH = plsc.VectorSubcoreMesh(core_axis_name="c", subcore_axis_name="s")
_SC_PARAMS = pltpu.CompilerParams(needs_layout_passes=False)
_HI = jax.lax.Precision.HIGHEST


def _adj_body(src_hbm, dst_hbm, uv_hbm, out_hbm,
              uv_v, den_v, src_v, dst_v, ex_v, att_v,
              didx, aidx, zbuf, den_sh, adj_sh,
              sem_stage, sem_zero, sem_scat):
    c = lax.axis_index("c")
    s = lax.axis_index("s")

    ebase = s * E_PER_SUB
    d_uv = pltpu.async_copy(uv_hbm, uv_v, sem_stage)
    d_src = pltpu.async_copy(src_hbm.at[pl.ds(ebase, E_PER_SUB)],
                             src_v.at[pl.ds(0, E_PER_SUB)], sem_stage)
    d_dst = pltpu.async_copy(dst_hbm.at[pl.ds(ebase, E_PER_SUB)],
                             dst_v.at[pl.ds(0, E_PER_SUB)], sem_stage)

    z16 = jnp.zeros((16,), F32)

    def _zb(i, _):
        zbuf[pl.ds(i * 16, 16)] = z16
        return None
    lax.fori_loop(0, ADJ_ZCHUNK // 16, _zb, None)

    zds = [pltpu.async_copy(
        zbuf,
        adj_sh.at[pl.ds(s * ADJ_WORDS_PER_SUB + t * ADJ_ZCHUNK, ADJ_ZCHUNK)],
        sem_zero) for t in range(8)]

    @pl.when(s == 0)
    def _():
        pltpu.sync_copy(zbuf.at[pl.ds(0, NP)], den_sh)

    plsc.subcore_barrier()

    d_uv.wait()
    d_src.wait()
    d_dst.wait()

    pad16 = jnp.full((16,), NP - 1, I32)
    for t in range(3):
        src_v[pl.ds(E_PER_SUB + t * 16, 16)] = pad16
        dst_v[pl.ds(E_PER_SUB + t * 16, 16)] = pad16

    cbase = c * ROWS_PER_CORE

    p1 = []
    for J in range(ECHUNKS):
        def _p1(i, _):
            o = J * 128 + i * 16
            s16 = src_v[pl.ds(o, 16)] & (NP - 1)
            d16 = dst_v[pl.ds(o, 16)] & (NP - 1)
            us = plsc.load_gather(uv_v, [s16 + s16])
            vs = plsc.load_gather(uv_v, [d16 + d16 + 1])
            e = us + vs
            e = jnp.where(e < 0.0, e * 0.2, e)
            ex_v[pl.ds(o, 16)] = jnp.exp(e)
            lidx = d16 - cbase
            inr = (lidx >= 0) & (lidx < ROWS_PER_CORE)
            dn = jnp.where(inr, lidx,
                           ROWS_PER_CORE + lax.shift_right_logical(s16, 1))
            didx[J, pl.ds(i * 16, 16)] = dn
            arow = jnp.where(inr, lidx,
                             ROWS_PER_CORE + lax.shift_right_logical(s16, 6))
            aidx[J, pl.ds(i * 16, 16)] = arow * NP + s16
            return None
        lax.fori_loop(0, 8, _p1, None)
        p1.append(pltpu.async_copy(ex_v.at[pl.ds(J * 128, 128)],
                                   den_sh.at[didx.at[J]], sem_scat, add=True))

    for d in zds:
        d.wait()
    for d in p1:
        d.wait()
    plsc.subcore_barrier()

    pltpu.sync_copy(den_sh, den_v)
    p2 = []
    for J in range(ECHUNKS):
        def _p2(i, _):
            o = J * 128 + i * 16
            dn = didx[J, pl.ds(i * 16, 16)]
            dv = plsc.load_gather(den_v, [dn])
            att_v[pl.ds(o, 16)] = ex_v[pl.ds(o, 16)] / (dv + 1e-16)
            return None
        lax.fori_loop(0, 8, _p2, None)
        p2.append(pltpu.async_copy(att_v.at[pl.ds(J * 128, 128)],
                                   adj_sh.at[aidx.at[J]], sem_scat, add=True))
    for d in p2:
        d.wait()
    plsc.subcore_barrier()

    ods = []
    for k in range(32):
        lr = s * 32 + k
        ods.append(pltpu.async_copy(adj_sh.at[pl.ds(lr * NP, NP)],
                                    out_hbm.at[c * ROWS_PER_CORE + lr],
                                    sem_stage))
    for d in ods:
        d.wait()


_adj_call = functools.partial(
    pl.kernel,
    out_type=jax.ShapeDtypeStruct((NP, NP), F32),
    mesh=_MESH,
    compiler_params=_SC_PARAMS,
    scratch_types=[
        pltpu.VMEM((2 * NP,), F32),
        pltpu.VMEM((NP,), F32),
        pltpu.VMEM((EDGES_PER_SUB,), I32),
        pltpu.VMEM((EDGES_PER_SUB,), I32),
        pltpu.VMEM((EDGES_PER_SUB,), F32),
        pltpu.VMEM((EDGES_PER_SUB,), F32),
        pltpu.VMEM((ECHUNKS, 128), I32),
        pltpu.VMEM((ECHUNKS, 128), I32),
        pltpu.VMEM((ADJ_ZCHUNK,), F32),
        pltpu.VMEM_SHARED((NP,), F32),
        pltpu.VMEM_SHARED((ADJ_WORDS,), F32),
        pltpu.SemaphoreType.DMA,
        pltpu.SemaphoreType.DMA,
        pltpu.SemaphoreType.DMA,
    ],
)(_adj_body)


def _abuild_body(cate_hbm, wp_hbm, out_hbm, wp_v, cl_v, idxb, zbuf, a_sh,
                 sem_stage, sem_zero, sem_scat):
    c = lax.axis_index("c")
    s = lax.axis_index("s")

    d_wp = pltpu.async_copy(wp_hbm, wp_v.at[pl.ds(0, N)], sem_stage)
    rows0 = c * (NS * A_ROWS_PER_SUB) + s * A_ROWS_PER_SUB
    d_cl = pltpu.async_copy(cate_hbm.at[pl.ds(rows0, A_ROWS_PER_SUB)], cl_v,
                            sem_stage)

    z16 = jnp.zeros((16,), F32)

    def _zb(i, _):
        zbuf[pl.ds(i * 16, 16)] = z16
        return None
    lax.fori_loop(0, A_ZCHUNK // 16, _zb, None)

    zds = [pltpu.async_copy(
        zbuf, a_sh.at[pl.ds(s * A_WORDS_PER_SUB + t * A_ZCHUNK, A_ZCHUNK)],
        sem_zero) for t in range(4)]

    d_wp.wait()
    d_cl.wait()
    for d in zds:
        d.wait()

    wp_v[pl.ds(1008, 16)] = z16
    lanes = lax.iota(I32, 16) + 992
    w16 = wp_v[pl.ds(992, 16)]
    wp_v[pl.ds(992, 16)] = jnp.where(lanes < N, w16, 0.0)

    for r in range(A_ROWS_PER_SUB):
        rbase = s * A_WORDS_PER_SUB + r * NP
        for j in range(8):
            def _fill(i, _):
                cv = cl_v[r, pl.ds(j * 128 + i * 16, 16)] & (NP - 1)
                idxb[r * 8 + j, pl.ds(i * 16, 16)] = cv + rbase
                return None
            lax.fori_loop(0, 8, _fill, None)

    descs = []
    for r in range(A_ROWS_PER_SUB):
        for j in range(8):
            descs.append(pltpu.async_copy(wp_v.at[pl.ds(j * 128, 128)],
                                          a_sh.at[idxb.at[r * 8 + j]],
                                          sem_scat, add=True))
        if r >= 1:
            for d in descs[(r - 1) * 8:r * 8]:
                d.wait()
    for d in descs[(A_ROWS_PER_SUB - 1) * 8:]:
        d.wait()

    ods = []
    for r in range(A_ROWS_PER_SUB):
        ods.append(pltpu.async_copy(
            a_sh.at[pl.ds((s * A_ROWS_PER_SUB + r) * NP, NP)],
            out_hbm.at[rows0 + r], sem_stage))
    for d in ods:
        d.wait()


_abuild_call = functools.partial(
    pl.kernel,
    out_type=jax.ShapeDtypeStruct((BATCH, NP), F32),
    mesh=_MESH,
    compiler_params=_SC_PARAMS,
    scratch_types=[
        pltpu.VMEM((NP,), F32),
        pltpu.VMEM((A_ROWS_PER_SUB, NP), I32),
        pltpu.VMEM((64, 128), I32),
        pltpu.VMEM((A_ZCHUNK,), F32),
        pltpu.VMEM_SHARED((A_WORDS_PER_CORE,), F32),
        pltpu.SemaphoreType.DMA,
        pltpu.SemaphoreType.DMA,
        pltpu.SemaphoreType.DMA,
    ],
)(_abuild_body)


def _dot(a, b):
    return jnp.dot(a, b, precision=_HI, preferred_element_type=F32)


def _hops_ln(adj, ht, h, g, b):
    bf = jnp.bfloat16
    ah = adj.astype(bf)
    al = (adj - ah.astype(F32)).astype(bf)

    def _mm3(zz):
        zh = zz.astype(bf)
        zl = (zz - zh.astype(F32)).astype(bf)
        acc = jnp.dot(ah, zh, preferred_element_type=F32)
        acc += jnp.dot(ah, zl, preferred_element_type=F32)
        acc += jnp.dot(al, zh, preferred_element_type=F32)
        return acc

    z = ht
    for _ in range(HOPS):
        z = (1.0 - ALPHA) * _mm3(z) + ALPHA * ht
    o = h + z
    mu = jnp.mean(o, axis=-1, keepdims=True)
    var = jnp.mean((o - mu) * (o - mu), axis=-1, keepdims=True)
    ln = g * (o - mu) * lax.rsqrt(var + 1e-5) + b
    rows = lax.broadcasted_iota(I32, (NP, HID), 0)
    return jnp.where(rows < N, ln, 0.0)


def _pad_rows(h):
    return jnp.concatenate([h, jnp.zeros((NP - N, HID), F32)], axis=0)


def _prep_body(h_ref, w_ref, a2_ref, ht_ref, uv_ref):
    ht = _dot(_pad_rows(h_ref[...]), w_ref[...])
    ht_ref[...] = ht
    uv_ref[...] = _dot(ht, a2_ref[...])


def _prep(h, w, a2):
    return pl.pallas_call(
        _prep_body,
        out_shape=[jax.ShapeDtypeStruct((NP, HID), F32),
                   jax.ShapeDtypeStruct((NP, 2), F32)],
    )(h, w, a2)


def _mid_body(adj_ref, ht_ref, h_ref, g_ref, b_ref, w2_ref, a2_ref,
              h2_ref, ht2_ref, uv2_ref):
    h2 = _hops_ln(adj_ref[...], ht_ref[...], _pad_rows(h_ref[...]),
                  g_ref[...], b_ref[...])
    h2_ref[...] = h2
    ht2 = _dot(h2, w2_ref[...])
    ht2_ref[...] = ht2
    uv2_ref[...] = _dot(ht2, a2_ref[...])


def _mid(adj, ht, h, g, b, w2, a2):
    return pl.pallas_call(
        _mid_body,
        out_shape=[jax.ShapeDtypeStruct((NP, HID), F32),
                   jax.ShapeDtypeStruct((NP, HID), F32),
                   jax.ShapeDtypeStruct((NP, 2), F32)],
    )(adj, ht, h, g, b, w2, a2)


def _tail_body(adj_ref, ht_ref, h_ref, g_ref, b_ref, wg_ref, bg_ref,
               gam_ref, bet_ref, wp_ref, bp_ref, a_ref, o_ref):
    gph = _hops_ln(adj_ref[...], ht_ref[...], h_ref[...], g_ref[...],
                   b_ref[...])
    logits = _dot(gph, wg_ref[...]) + bg_ref[...]
    mx = jnp.max(logits, axis=1, keepdims=True)
    exl = jnp.exp(logits - mx)
    score = exl / jnp.sum(exl, axis=1, keepdims=True)

    mu = jnp.sum(gph, axis=0, keepdims=True) / float(NG * N)
    s2 = jnp.sum(score * score, axis=1, keepdims=True)
    msq = jnp.sum(gph * gph * s2, axis=0, keepdims=True) / float(NG * N)
    rstd = lax.rsqrt(msq - mu * mu + 1e-5)

    scale = gam_ref[...] * rstd
    rows = lax.broadcasted_iota(I32, (NP, 1), 0)
    keep = (rows > 0) & (rows < N)
    m = jnp.where(keep, gph * scale, 0.0)
    mp = jnp.concatenate([score[:, g:g + 1] * m for g in range(NG)], axis=1)

    sw = jnp.sum(wp_ref[...])
    cvec = (bet_ref[...] - gam_ref[...] * mu * rstd) * sw + bp_ref[...]
    cv = jnp.concatenate([cvec] * NG, axis=1)
    o_ref[...] = _dot(a_ref[...], mp) + cv


def _tail(adj, ht, h, g, b, wg, bg, gam, bet, wp, bp, a):
    return pl.pallas_call(
        _tail_body,
        out_shape=jax.ShapeDtypeStruct((BATCH, NG * HID), F32),
    )(adj, ht, h, g, b, wg, bg, gam, bet, wp, bp, a)


def kernel(cate_list, edge_index, emb, W1, a_src1, a_dst1, ln_g1, ln_b1,
           W2, a_src2, a_dst2, ln_g2, ln_b2, Wg, bg, Wpool, bpool,
           bn_gamma, bn_beta):
    cate_p = jnp.pad(cate_list, ((0, 0), (0, NP - N)))
    a_mat = _abuild_call(cate_p, Wpool)

    src = edge_index[0]
    dst = edge_index[1]
    ht1, uv1 = _prep(emb, W1, jnp.stack([a_src1, a_dst1], axis=1))
    adj1 = _adj_call(src, dst, uv1.reshape(-1))
    h2, ht2, uv2 = _mid(adj1, ht1, emb,
                        ln_g1.reshape(1, HID), ln_b1.reshape(1, HID),
                        W2, jnp.stack([a_src2, a_dst2], axis=1))

    adj2 = _adj_call(src, dst, uv2.reshape(-1))
    out = _tail(adj2, ht2, h2,
                ln_g2.reshape(1, HID), ln_b2.reshape(1, HID),
                Wg, bg.reshape(1, NG), bn_gamma.reshape(1, HID),
                bn_beta.reshape(1, HID), Wpool.reshape(1, N),
                bpool.reshape(1, 1), a_mat)
    return out.reshape(BATCH, NG, HID)
